# Initial kernel scaffold; baseline (speedup 1.0000x reference)
import jax
import jax.numpy as jnp
from jax.experimental import pallas as pl


def kernel(x_nchw, w_oihw, gamma, beta):
    raise NotImplementedError("write your pallas kernel here")


# speedup vs baseline: 4.5586x; 4.5586x over previous
import functools

import jax
import jax.numpy as jnp
from jax.experimental import pallas as pl
from jax.experimental.pallas import tpu as pltpu

_BN_EPS = 1e-5
_VMEM_LIMIT = 32 * 1024 * 1024


def _build_patches(x3, xc_ref, H, W, Cin):
    R = H * W
    for kh in range(3):
        for kw in range(3):
            t = kh * 3 + kw
            v = x3[kh:kh + H, kw:kw + W, :].reshape(R, Cin)
            xc_ref[:, t * Cin:(t + 1) * Cin] = v


def _stats_kernel(H, W, Cin, x_ref, w_ref, stats_ref, xc_ref):
    x3 = x_ref[0]
    _build_patches(x3, xc_ref, H, W, Cin)
    y = jnp.dot(xc_ref[...], w_ref[...], preferred_element_type=jnp.float32)
    stats_ref[0, 0, :] = jnp.sum(y, axis=0)
    stats_ref[0, 1, :] = jnp.sum(y * y, axis=0)


def _out_kernel(H, W, Cin, x_ref, w_ref, scale_ref, shift_ref, o_ref, xc_ref):
    x3 = x_ref[0]
    _build_patches(x3, xc_ref, H, W, Cin)
    yt = jax.lax.dot_general(
        w_ref[...], xc_ref[...],
        dimension_numbers=(((0,), (1,)), ((), ())),
        preferred_element_type=jnp.float32)
    o_ref[0] = jnp.maximum(yt * scale_ref[...] + shift_ref[...], 0.0)


def kernel(x_nchw, w_oihw, gamma, beta):
    N, Cin, H, W = x_nchw.shape
    Cout = w_oihw.shape[0]
    K = 9 * Cin
    R = H * W

    x_nhwc = jnp.transpose(x_nchw, (0, 2, 3, 1)).astype(jnp.bfloat16)
    xp = jnp.pad(x_nhwc, ((0, 0), (1, 1), (1, 1), (0, 0)))
    w_mat = jnp.transpose(w_oihw, (2, 3, 1, 0)).reshape(K, Cout).astype(jnp.bfloat16)

    params = pltpu.CompilerParams(
        dimension_semantics=("parallel",),
        vmem_limit_bytes=_VMEM_LIMIT)

    stats = pl.pallas_call(
        functools.partial(_stats_kernel, H, W, Cin),
        out_shape=jax.ShapeDtypeStruct((N, 2, Cout), jnp.float32),
        grid=(N,),
        in_specs=[
            pl.BlockSpec((1, H + 2, W + 2, Cin), lambda i: (i, 0, 0, 0)),
            pl.BlockSpec((K, Cout), lambda i: (0, 0)),
        ],
        out_specs=pl.BlockSpec((1, 2, Cout), lambda i: (i, 0, 0)),
        scratch_shapes=[pltpu.VMEM((R, K), jnp.bfloat16)],
        compiler_params=params,
    )(xp, w_mat)

    tot = jnp.sum(stats, axis=0)
    cnt = jnp.float32(N * R)
    mean = tot[0] / cnt
    var = tot[1] / cnt - mean * mean
    inv_std = jax.lax.rsqrt(var + _BN_EPS)
    scale = (gamma.astype(jnp.float32) * inv_std).reshape(Cout, 1)
    shift = (beta.astype(jnp.float32) - mean * gamma.astype(jnp.float32)
             * inv_std).reshape(Cout, 1)

    out_flat = pl.pallas_call(
        functools.partial(_out_kernel, H, W, Cin),
        out_shape=jax.ShapeDtypeStruct((N, Cout, R), jnp.float32),
        grid=(N,),
        in_specs=[
            pl.BlockSpec((1, H + 2, W + 2, Cin), lambda i: (i, 0, 0, 0)),
            pl.BlockSpec((K, Cout), lambda i: (0, 0)),
            pl.BlockSpec((Cout, 1), lambda i: (0, 0)),
            pl.BlockSpec((Cout, 1), lambda i: (0, 0)),
        ],
        out_specs=pl.BlockSpec((1, Cout, R), lambda i: (i, 0, 0)),
        scratch_shapes=[pltpu.VMEM((R, K), jnp.bfloat16)],
        compiler_params=params,
    )(xp, w_mat, scale, shift)

    return out_flat.reshape(N, Cout, H, W)
